# baseline (speedup 1.0000x reference)
import functools
import math

import jax
import jax.numpy as jnp
from jax import lax
from jax.experimental import pallas as pl
from jax.experimental.pallas import tpu as pltpu
from jax.experimental.pallas import tpu_sc as plsc

VOCAB = 1000000
D_MODEL = 64
SCALE = math.sqrt(D_MODEL)

_INFO = plsc.get_sparse_core_info()
NC, NS, L = _INFO.num_cores, _INFO.num_subcores, _INFO.num_lanes
NW = NC * NS

BATCH = 4096
SEQ = 50
BT = 128
NBT = BATCH // BT
N_UNITS = SEQ * NBT
U_PER_W = N_UNITS // NW
FT = D_MODEL // 8

PBLK = 512
NPBLK = -(-VOCAB // (2 * PBLK)) + 0
H = NPBLK * PBLK


def _pack_body(a_ref, b_ref, o_ref):
    z = jnp.concatenate([a_ref[...], b_ref[...]], axis=0)
    o_ref[...] = z.T


def _tc_pack(tt):
    return pl.pallas_call(
        _pack_body,
        grid=(NPBLK,),
        in_specs=[
            pl.BlockSpec((D_MODEL, PBLK), lambda g: (0, g)),
            pl.BlockSpec((D_MODEL, PBLK), lambda g: (0, g + NPBLK)),
        ],
        out_specs=pl.BlockSpec((PBLK, 2 * D_MODEL), lambda g: (g, 0)),
        out_shape=jax.ShapeDtypeStruct((H, 2 * D_MODEL), jnp.float32),
    )(tt, tt)


def _sc_body(xt_hbm, tbl_hbm, out_hbm, idx_v, g0, g1, o0, o1,
             gs0, gs1, os0, os1):
    gbufs, obufs = (g0, g1), (o0, o1)
    gsems, osems = (gs0, gs1), (os0, os1)

    wid = lax.axis_index("s") * NC + lax.axis_index("c")
    u0 = wid * U_PER_W

    pltpu.sync_copy(xt_hbm.at[pl.ds(u0, U_PER_W)], idx_v)

    def idx_step(k, _):
        for g in range(BT // L):
            sl = pl.ds(L * g, L)
            v = idx_v[k, sl]
            wrap = jnp.where(v >= H, 2 * H - 1, 0)
            idx_v[k, sl] = v + v - wrap
        return 0

    lax.fori_loop(0, U_PER_W, idx_step, 0, unroll=2)

    def start_gather(k, b):
        pltpu.async_copy(tbl_hbm.at[idx_v.at[k]], gbufs[b], gsems[b])

    def wait_gather(k, b):
        pltpu.make_async_copy(tbl_hbm.at[idx_v.at[k]], gbufs[b],
                              gsems[b]).wait()

    def start_out(k, b):
        u = u0 + k
        s = u // NBT
        bt = lax.rem(u, NBT)
        for ft in range(FT):
            pltpu.async_copy(obufs[b].at[pl.ds(8 * ft, 8), pl.ds(0, BT)],
                             out_hbm.at[s, ft, bt], osems[b])

    def wait_out(b):
        for ft in range(FT):
            pltpu.make_async_copy(obufs[b].at[pl.ds(8 * ft, 8), pl.ds(0, BT)],
                                  out_hbm.at[0, 0, 0], osems[b]).wait()

    feat_ids = [lax.iota(jnp.int32, L) + L * c for c in range(D_MODEL // L)]

    def transpose_scale(b):
        gb, ob = gbufs[b], obufs[b]

        def row_step(r, _):
            rvec = jnp.zeros((L,), jnp.int32) + r
            for c in range(D_MODEL // L):
                v = gb[r, pl.ds(L * c, L)]
                plsc.store_scatter(ob, [feat_ids[c], rvec], v * SCALE)
            return 0

        lax.fori_loop(0, BT, row_step, 0, unroll=2)

    start_gather(0, 0)
    start_gather(1, 1)

    for b in range(2):
        wait_gather(b, b)
        transpose_scale(b)
        start_out(b, b)
        start_gather(b + 2, b)

    def pair(i, _):
        for b in range(2):
            k = 2 * i + b
            wait_gather(k, b)
            wait_out(b)
            transpose_scale(b)
            start_out(k, b)
            start_gather(k + 2, b)
        return 0

    lax.fori_loop(1, U_PER_W // 2 - 1, pair, 0)

    for b in range(2):
        k = U_PER_W - 2 + b
        wait_gather(k, b)
        wait_out(b)
        transpose_scale(b)
        start_out(k, b)

    for b in range(2):
        wait_out(b)


def kernel(x, table):
    xt = x.T.reshape(N_UNITS, BT).astype(jnp.int32)
    ppack = _tc_pack(table.T)
    p2 = ppack.reshape(2 * H, D_MODEL)
    mesh = plsc.VectorSubcoreMesh(core_axis_name="c", subcore_axis_name="s")
    scratch = [pltpu.VMEM((U_PER_W, BT), jnp.int32)]
    scratch += [pltpu.VMEM((BT, D_MODEL), jnp.float32) for _ in range(2)]
    scratch += [pltpu.VMEM((D_MODEL, BT + 1), jnp.float32) for _ in range(2)]
    scratch += [pltpu.SemaphoreType.DMA for _ in range(4)]
    sc_call = pl.kernel(
        _sc_body,
        mesh=mesh,
        out_type=jax.ShapeDtypeStruct((SEQ, FT, NBT, 8, BT), jnp.float32),
        scratch_types=scratch,
        compiler_params=pltpu.CompilerParams(use_tc_tiling_on_sc=False,
                                             needs_layout_passes=False),
    )
    out5 = sc_call(xt, p2)
    return out5.transpose(2, 4, 0, 1, 3).reshape(BATCH, SEQ, D_MODEL)


# speedup vs baseline: 1.2060x; 1.2060x over previous
import functools
import math

import jax
import jax.numpy as jnp
from jax import lax
from jax.experimental import pallas as pl
from jax.experimental.pallas import tpu as pltpu
from jax.experimental.pallas import tpu_sc as plsc

VOCAB = 1000000
D_MODEL = 64
SCALE = math.sqrt(D_MODEL)

_INFO = plsc.get_sparse_core_info()
NC, NS, L = _INFO.num_cores, _INFO.num_subcores, _INFO.num_lanes
NW = NC * NS

BATCH = 4096
SEQ = 50
BT = 128
NBT = BATCH // BT
N_UNITS = SEQ * NBT
U_PER_W = N_UNITS // NW
FT = D_MODEL // 8

PBLK = 512
NPBLK = -(-VOCAB // (2 * PBLK)) + 0
H = NPBLK * PBLK


def _pack_body(a_ref, b_ref, o_ref):
    z = jnp.concatenate([a_ref[...], b_ref[...]], axis=0)
    o_ref[...] = z.T


def _tc_pack(tt):
    return pl.pallas_call(
        _pack_body,
        grid=(NPBLK,),
        in_specs=[
            pl.BlockSpec((D_MODEL, PBLK), lambda g: (0, g)),
            pl.BlockSpec((D_MODEL, PBLK), lambda g: (0, g + NPBLK)),
        ],
        out_specs=pl.BlockSpec((PBLK, 2 * D_MODEL), lambda g: (g, 0)),
        out_shape=jax.ShapeDtypeStruct((H, 2 * D_MODEL), jnp.float32),
    )(tt, tt)


def _sc_body(xt_hbm, tbl_hbm, out_hbm, idx_v, g0, g1, o0, o1,
             gs0, gs1, os0, os1):
    gbufs, obufs = (g0, g1), (o0, o1)
    gsems, osems = (gs0, gs1), (os0, os1)

    wid = lax.axis_index("s") * NC + lax.axis_index("c")
    u0 = wid * U_PER_W

    pltpu.sync_copy(xt_hbm.at[pl.ds(u0, U_PER_W)], idx_v)

    def idx_step(k, _):
        for g in range(BT // L):
            sl = pl.ds(L * g, L)
            v = idx_v[k, sl]
            wrap = jnp.where(v >= H, 2 * H - 1, 0)
            idx_v[k, sl] = v + v - wrap
        return 0

    lax.fori_loop(0, U_PER_W, idx_step, 0, unroll=2)

    def start_gather(k, b):
        pltpu.async_copy(tbl_hbm.at[idx_v.at[k]], gbufs[b], gsems[b])

    def wait_gather(k, b):
        pltpu.make_async_copy(tbl_hbm.at[idx_v.at[k]], gbufs[b],
                              gsems[b]).wait()

    def start_out(k, b):
        u = u0 + k
        s = u // NBT
        bt = lax.rem(u, NBT)
        for ft in range(FT):
            pltpu.async_copy(obufs[b].at[pl.ds(8 * ft, 8), pl.ds(0, BT)],
                             out_hbm.at[s, ft, bt], osems[b])

    def wait_out(b):
        for ft in range(FT):
            pltpu.make_async_copy(obufs[b].at[pl.ds(8 * ft, 8), pl.ds(0, BT)],
                                  out_hbm.at[0, 0, 0], osems[b]).wait()

    feat_ids = [lax.iota(jnp.int32, L) + L * c for c in range(D_MODEL // L)]

    def transpose_scale(b):
        gb, ob = gbufs[b], obufs[b]

        def row_step(r, _):
            rvec = jnp.zeros((L,), jnp.int32) + r
            for c in range(D_MODEL // L):
                v = gb[r, pl.ds(L * c, L)]
                plsc.store_scatter(ob, [feat_ids[c], rvec], v * SCALE)
            return 0

        lax.fori_loop(0, BT, row_step, 0, unroll=2)

    start_gather(0, 0)
    start_gather(1, 1)

    for b in range(2):
        wait_gather(b, b)
        transpose_scale(b)
        start_out(b, b)
        start_gather(b + 2, b)

    def pair(i, _):
        for b in range(2):
            k = 2 * i + b
            wait_gather(k, b)
            wait_out(b)
            transpose_scale(b)
            start_out(k, b)
            start_gather(k + 2, b)
        return 0

    lax.fori_loop(1, U_PER_W // 2 - 1, pair, 0)

    for b in range(2):
        k = U_PER_W - 2 + b
        wait_gather(k, b)
        wait_out(b)
        transpose_scale(b)
        start_out(k, b)

    for b in range(2):
        wait_out(b)


def kernel(x, table):
    xt = x.T.reshape(N_UNITS, BT).astype(jnp.int32)
    ppack = jnp.concatenate(
        [table[:H], jnp.pad(table[H:], ((0, 2 * H - VOCAB), (0, 0)))],
        axis=1)
    p2 = ppack.reshape(2 * H, D_MODEL)
    mesh = plsc.VectorSubcoreMesh(core_axis_name="c", subcore_axis_name="s")
    scratch = [pltpu.VMEM((U_PER_W, BT), jnp.int32)]
    scratch += [pltpu.VMEM((BT, D_MODEL), jnp.float32) for _ in range(2)]
    scratch += [pltpu.VMEM((D_MODEL, BT + 1), jnp.float32) for _ in range(2)]
    scratch += [pltpu.SemaphoreType.DMA for _ in range(4)]
    sc_call = pl.kernel(
        _sc_body,
        mesh=mesh,
        out_type=jax.ShapeDtypeStruct((SEQ, FT, NBT, 8, BT), jnp.float32),
        scratch_types=scratch,
        compiler_params=pltpu.CompilerParams(use_tc_tiling_on_sc=False,
                                             needs_layout_passes=False),
    )
    out5 = sc_call(xt, p2)
    return out5.transpose(2, 4, 0, 1, 3).reshape(BATCH, SEQ, D_MODEL)
